# Initial kernel scaffold; baseline (speedup 1.0000x reference)
import jax
import jax.numpy as jnp
from jax.experimental import pallas as pl


def kernel(inputs, mapping):
    raise NotImplementedError("write your pallas kernel here")


# speedup vs baseline: 201.4435x; 201.4435x over previous
import functools

import jax
import jax.numpy as jnp
from jax import lax
from jax.experimental import pallas as pl
from jax.experimental.pallas import tpu as pltpu
from jax.experimental.pallas import tpu_sc as plsc

ROWS, COLS = 16384, 200
N = ROWS * COLS
NC, NS = 2, 16
NW = NC * NS
PER_W = N // NW
CHUNK = 12800
NCHUNK = PER_W // CHUNK
LANES = 16


def _body(in_hbm, map_hbm, out_hbm, table_v, in_v, out_v):
    wid = lax.axis_index("s") * NC + lax.axis_index("c")
    base = wid * PER_W
    pltpu.sync_copy(map_hbm, table_v)

    def chunk_body(ci, _):
        off = base + ci * CHUNK
        pltpu.sync_copy(in_hbm.at[pl.ds(off, CHUNK)], in_v)

        def gather_step(i, _):
            idx = in_v[pl.ds(i * LANES, LANES)]
            out_v[pl.ds(i * LANES, LANES)] = plsc.load_gather(table_v, [idx])
            return 0

        lax.fori_loop(0, CHUNK // LANES, gather_step, 0, unroll=8)
        pltpu.sync_copy(out_v, out_hbm.at[pl.ds(off, CHUNK)])
        return 0

    lax.fori_loop(0, NCHUNK, chunk_body, 0)


@jax.jit
def _lookup(inputs_flat, mapping):
    mesh = plsc.VectorSubcoreMesh(core_axis_name="c", subcore_axis_name="s")
    run = pl.kernel(
        _body,
        out_type=jax.ShapeDtypeStruct((N,), jnp.int32),
        mesh=mesh,
        scratch_types=[
            pltpu.VMEM((256,), jnp.int32),
            pltpu.VMEM((CHUNK,), jnp.int32),
            pltpu.VMEM((CHUNK,), jnp.int32),
        ],
        compiler_params=pltpu.CompilerParams(needs_layout_passes=False),
    )
    return run(inputs_flat, mapping)


def kernel(inputs, mapping):
    out_flat = _lookup(inputs.reshape(N), mapping)
    return out_flat.reshape(ROWS, COLS)


# speedup vs baseline: 280.2171x; 1.3910x over previous
import functools

import jax
import jax.numpy as jnp
from jax import lax
from jax.experimental import pallas as pl
from jax.experimental.pallas import tpu as pltpu
from jax.experimental.pallas import tpu_sc as plsc

ROWS, COLS = 16384, 200
N = ROWS * COLS
NC, NS = 2, 16
NW = NC * NS
PER_W = N // NW
CHUNK = 12800
NCHUNK = PER_W // CHUNK
LANES = 16


def _body(in_hbm, map_hbm, out_hbm, table_v, in_v, out_v):
    wid = lax.axis_index("s") * NC + lax.axis_index("c")
    base = wid * PER_W
    pltpu.sync_copy(map_hbm, table_v)

    def chunk_body(ci, _):
        off = base + ci * CHUNK
        pltpu.sync_copy(in_hbm.at[pl.ds(off, CHUNK)], in_v)

        @plsc.parallel_loop(0, CHUNK // LANES, unroll=8)
        def gather_step(i):
            idx = in_v[pl.ds(i * LANES, LANES)]
            out_v[pl.ds(i * LANES, LANES)] = plsc.load_gather(table_v, [idx])
        pltpu.sync_copy(out_v, out_hbm.at[pl.ds(off, CHUNK)])
        return 0

    lax.fori_loop(0, NCHUNK, chunk_body, 0)


@jax.jit
def _lookup(inputs_flat, mapping):
    mesh = plsc.VectorSubcoreMesh(core_axis_name="c", subcore_axis_name="s")
    run = pl.kernel(
        _body,
        out_type=jax.ShapeDtypeStruct((N,), jnp.int32),
        mesh=mesh,
        scratch_types=[
            pltpu.VMEM((256,), jnp.int32),
            pltpu.VMEM((CHUNK,), jnp.int32),
            pltpu.VMEM((CHUNK,), jnp.int32),
        ],
        compiler_params=pltpu.CompilerParams(needs_layout_passes=False),
    )
    return run(inputs_flat, mapping)


def kernel(inputs, mapping):
    out_flat = _lookup(inputs.reshape(N), mapping)
    return out_flat.reshape(ROWS, COLS)


# speedup vs baseline: 304.5120x; 1.0867x over previous
import jax
import jax.numpy as jnp
from jax import lax
from jax.experimental import pallas as pl
from jax.experimental.pallas import tpu as pltpu
from jax.experimental.pallas import tpu_sc as plsc

ROWS, COLS = 16384, 200
N = ROWS * COLS
NC, NS = 2, 16
NW = NC * NS
PER_W = N // NW
CHUNK = 12800
NCHUNK = PER_W // CHUNK
LANES = 16
UNROLL = 8


def _body(in_hbm, map_hbm, out_hbm, table_v,
          in_a, in_b, out_a, out_b,
          sem_ia, sem_ib, sem_oa, sem_ob):
    wid = lax.axis_index("s") * NC + lax.axis_index("c")
    base = wid * PER_W
    pltpu.sync_copy(map_hbm, table_v)

    in_bufs = (in_a, in_b)
    out_bufs = (out_a, out_b)
    in_sems = (sem_ia, sem_ib)
    out_sems = (sem_oa, sem_ob)

    def start_in(k):
        return pltpu.async_copy(
            in_hbm.at[pl.ds(base + k * CHUNK, CHUNK)],
            in_bufs[k % 2], in_sems[k % 2])

    def start_out(k):
        return pltpu.async_copy(
            out_bufs[k % 2],
            out_hbm.at[pl.ds(base + k * CHUNK, CHUNK)],
            out_sems[k % 2])

    def compute(k):
        src = in_bufs[k % 2]
        dst = out_bufs[k % 2]

        @plsc.parallel_loop(0, CHUNK // LANES, unroll=UNROLL)
        def gather_step(i):
            idx = src[pl.ds(i * LANES, LANES)]
            dst[pl.ds(i * LANES, LANES)] = plsc.load_gather(table_v, [idx])

    in_dma = [None] * NCHUNK
    out_dma = [None] * NCHUNK
    in_dma[0] = start_in(0)
    in_dma[1] = start_in(1)
    for k in range(NCHUNK):
        in_dma[k].wait()
        if k >= 2:
            out_dma[k - 2].wait()
        compute(k)
        out_dma[k] = start_out(k)
        if k + 2 < NCHUNK:
            in_dma[k + 2] = start_in(k + 2)
    out_dma[NCHUNK - 2].wait()
    out_dma[NCHUNK - 1].wait()


@jax.jit
def _lookup(inputs_flat, mapping):
    mesh = plsc.VectorSubcoreMesh(core_axis_name="c", subcore_axis_name="s")
    run = pl.kernel(
        _body,
        out_type=jax.ShapeDtypeStruct((N,), jnp.int32),
        mesh=mesh,
        scratch_types=[
            pltpu.VMEM((256,), jnp.int32),
            pltpu.VMEM((CHUNK,), jnp.int32),
            pltpu.VMEM((CHUNK,), jnp.int32),
            pltpu.VMEM((CHUNK,), jnp.int32),
            pltpu.VMEM((CHUNK,), jnp.int32),
            pltpu.SemaphoreType.DMA,
            pltpu.SemaphoreType.DMA,
            pltpu.SemaphoreType.DMA,
            pltpu.SemaphoreType.DMA,
        ],
        compiler_params=pltpu.CompilerParams(needs_layout_passes=False),
    )
    return run(inputs_flat, mapping)


def kernel(inputs, mapping):
    out_flat = _lookup(inputs.reshape(N), mapping)
    return out_flat.reshape(ROWS, COLS)
